# baseline (speedup 1.0000x reference)
import functools

import jax
import jax.numpy as jnp
import numpy as np
from jax import lax
from jax.experimental import pallas as pl
from jax.experimental.pallas import tpu as pltpu
from jax.experimental.pallas import tpu_sc as plsc

_D = 128
_SCALE = float(np.sqrt(_D))
_NC, _NS, _L = 2, 16, 16
_NW = _NC * _NS


def _pos_table(seq_len: int) -> np.ndarray:
    pos = np.arange(seq_len, dtype=np.float32)[:, None]
    i2 = np.arange(0, _D, 2, dtype=np.float32)
    ang = (pos / np.power(np.float32(10000.0), i2 / np.float32(_D))).astype(np.float32)
    enc = np.zeros((seq_len, _D), dtype=np.float32)
    enc[:, 0::2] = np.sin(ang)
    enc[:, 1::2] = np.cos(ang)
    return enc


def kernel(x, table):
    B, S = x.shape
    N = B * S
    b_per_w = N // _NW
    NCH = 4
    C = b_per_w // NCH
    assert N % _NW == 0 and b_per_w % NCH == 0 and S % b_per_w == 0

    pos_div = _pos_table(S) * np.float32(1.0 / _SCALE)
    xf = x.reshape(N)

    mesh = plsc.VectorSubcoreMesh(
        core_axis_name="c", subcore_axis_name="s",
        num_cores=_NC, num_subcores=_NS,
    )

    @functools.partial(
        pl.kernel,
        out_type=jax.ShapeDtypeStruct((N, _D), jnp.float32),
        mesh=mesh,
        scratch_types=[
            pltpu.VMEM((b_per_w,), jnp.int32),
            pltpu.VMEM((NCH, C, _D), jnp.float32),
            [pltpu.SemaphoreType.DMA] * NCH,
            [pltpu.SemaphoreType.DMA] * NCH,
            [pltpu.SemaphoreType.DMA] * NCH,
        ],
    )
    def emb_kernel(x_hbm, table_hbm, pos_hbm, out_hbm,
                   idx_v, rows_v, psems, gsems, wsems):
        wid = lax.axis_index("s") * _NC + lax.axis_index("c")
        base = wid * b_per_w
        pos_base = lax.rem(base, S)

        pltpu.sync_copy(x_hbm.at[pl.ds(base, b_per_w)], idx_v)
        pos_descs = [
            pltpu.async_copy(pos_hbm.at[pl.ds(pos_base + c * C, C)],
                             rows_v.at[c], psems[c])
            for c in range(NCH)
        ]
        gadd_descs = []
        for c in range(NCH):
            pos_descs[c].wait()
            gadd_descs.append(
                pltpu.async_copy(table_hbm.at[idx_v.at[pl.ds(c * C, C)]],
                                 rows_v.at[c], gsems[c], add=True))
        wb_descs = []
        for c in range(NCH):
            gadd_descs[c].wait()

            def _make_scale(c):
                @plsc.parallel_loop(0, C, unroll=4)
                def _scale(i):
                    for j in range(_D // _L):
                        sl = pl.ds(j * _L, _L)
                        rows_v[c, i, sl] = rows_v[c, i, sl] * _SCALE

            _make_scale(c)

            wb_descs.append(
                pltpu.async_copy(rows_v.at[c],
                                 out_hbm.at[pl.ds(base + c * C, C)],
                                 wsems[c]))
        for c in range(NCH):
            wb_descs[c].wait()

    out = emb_kernel(xf, table, pos_div)
    return out.reshape(B, S, _D)


# speedup vs baseline: 1.0686x; 1.0686x over previous
import functools

import jax
import jax.numpy as jnp
import numpy as np
from jax import lax
from jax.experimental import pallas as pl
from jax.experimental.pallas import tpu as pltpu
from jax.experimental.pallas import tpu_sc as plsc

_D = 128
_SCALE = float(np.sqrt(_D))
_NC, _NS, _L = 2, 16, 16
_NW = _NC * _NS


def _pos_table(seq_len: int) -> np.ndarray:
    pos = np.arange(seq_len, dtype=np.float32)[:, None]
    i2 = np.arange(0, _D, 2, dtype=np.float32)
    ang = (pos / np.power(np.float32(10000.0), i2 / np.float32(_D))).astype(np.float32)
    enc = np.zeros((seq_len, _D), dtype=np.float32)
    enc[:, 0::2] = np.sin(ang)
    enc[:, 1::2] = np.cos(ang)
    return enc


def kernel(x, table):
    B, S = x.shape
    N = B * S
    C = S // _NW
    assert S % _NW == 0 and _D % _L == 0

    pos = _pos_table(S)
    xw = x.reshape(B, _NW, C).transpose(1, 0, 2)

    mesh = plsc.VectorSubcoreMesh(
        core_axis_name="c", subcore_axis_name="s",
        num_cores=_NC, num_subcores=_NS,
    )

    @functools.partial(
        pl.kernel,
        out_type=jax.ShapeDtypeStruct((N, _D), jnp.float32),
        mesh=mesh,
        scratch_types=[
            pltpu.VMEM((B, C), jnp.int32),
            pltpu.VMEM((C, _D), jnp.float32),
            pltpu.VMEM((B, C, _D), jnp.float32),
            pltpu.SemaphoreType.DMA,
            [pltpu.SemaphoreType.DMA] * 4,
            [pltpu.SemaphoreType.DMA] * 4,
        ],
    )
    def emb_kernel(x_hbm, table_hbm, pos_hbm, out_hbm,
                   idx_v, pos_v, rows_v, psem, gsems, wsems):
        wid = lax.axis_index("s") * _NC + lax.axis_index("c")
        ws = wid * C

        pltpu.sync_copy(x_hbm.at[wid], idx_v)
        pos_desc = pltpu.async_copy(pos_hbm.at[pl.ds(ws, C)], pos_v, psem)
        gath_descs = [
            pltpu.async_copy(table_hbm.at[idx_v.at[c]], rows_v.at[c],
                             gsems[c])
            for c in range(B)
        ]
        pos_desc.wait()
        wb_descs = []
        for c in range(B):
            gath_descs[c].wait()

            def _make_scale(c):
                @plsc.parallel_loop(0, C, unroll=4)
                def _scale(i):
                    for j in range(_D // _L):
                        sl = pl.ds(j * _L, _L)
                        rows_v[c, i, sl] = (rows_v[c, i, sl] * _SCALE
                                            + pos_v[i, sl])

            _make_scale(c)
            wb_descs.append(
                pltpu.async_copy(rows_v.at[c],
                                 out_hbm.at[pl.ds(c * S + ws, C)],
                                 wsems[c]))
        for c in range(B):
            wb_descs[c].wait()

    out = emb_kernel(xw, table, pos)
    return out.reshape(B, S, _D)


# speedup vs baseline: 1.1147x; 1.0431x over previous
import functools

import jax
import jax.numpy as jnp
import numpy as np
from jax import lax
from jax.experimental import pallas as pl
from jax.experimental.pallas import tpu as pltpu
from jax.experimental.pallas import tpu_sc as plsc

_D = 128
_SCALE = float(np.sqrt(_D))
_NC, _NS, _L = 2, 16, 16
_NW = _NC * _NS


def _pos_table(seq_len: int) -> np.ndarray:
    pos = np.arange(seq_len, dtype=np.float32)[:, None]
    i2 = np.arange(0, _D, 2, dtype=np.float32)
    ang = (pos / np.power(np.float32(10000.0), i2 / np.float32(_D))).astype(np.float32)
    enc = np.zeros((seq_len, _D), dtype=np.float32)
    enc[:, 0::2] = np.sin(ang)
    enc[:, 1::2] = np.cos(ang)
    return enc


def kernel(x, table):
    B, S = x.shape
    N = B * S
    C = S // _NW
    assert S % _NW == 0 and _D % _L == 0

    pos = _pos_table(S)
    xw = x.reshape(B, _NW, C).transpose(1, 0, 2)

    mesh = plsc.VectorSubcoreMesh(
        core_axis_name="c", subcore_axis_name="s",
        num_cores=_NC, num_subcores=_NS,
    )

    @functools.partial(
        pl.kernel,
        out_type=jax.ShapeDtypeStruct((N, _D), jnp.float32),
        mesh=mesh,
        scratch_types=[
            pltpu.VMEM((B, C), jnp.int32),
            pltpu.VMEM((C, _D), jnp.float32),
            pltpu.VMEM((B, C, _D), jnp.float32),
            pltpu.SemaphoreType.DMA,
            [pltpu.SemaphoreType.DMA] * 4,
            [pltpu.SemaphoreType.DMA] * 4,
        ],
    )
    def emb_kernel(x_hbm, table_hbm, pos_hbm, out_hbm,
                   idx_v, pos_v, rows_v, psem, gsems, wsems):
        wid = lax.axis_index("s") * _NC + lax.axis_index("c")
        ws = wid * C

        pltpu.sync_copy(x_hbm.at[wid], idx_v)
        pos_desc = pltpu.async_copy(pos_hbm.at[pl.ds(ws, C)], pos_v, psem)
        gath_descs = [
            pltpu.async_copy(table_hbm.at[idx_v.at[c]], rows_v.at[c],
                             gsems[c])
            for c in range(B)
        ]
        pos_desc.wait()
        wb_descs = []
        for c0 in range(0, B, 2):
            c1 = c0 + 1
            gath_descs[c0].wait()
            gath_descs[c1].wait()

            def _make_scale(c0, c1):
                @plsc.parallel_loop(0, C, unroll=4)
                def _scale(i):
                    for j in range(_D // _L):
                        sl = pl.ds(j * _L, _L)
                        pv = pos_v[i, sl]
                        rows_v[c0, i, sl] = rows_v[c0, i, sl] * _SCALE + pv
                        rows_v[c1, i, sl] = rows_v[c1, i, sl] * _SCALE + pv

            _make_scale(c0, c1)
            for c in (c0, c1):
                wb_descs.append(
                    pltpu.async_copy(rows_v.at[c],
                                     out_hbm.at[pl.ds(c * S + ws, C)],
                                     wsems[c]))
        for d in wb_descs:
            d.wait()

    out = emb_kernel(xw, table, pos)
    return out.reshape(B, S, _D)
